# baseline (speedup 1.0000x reference)
import jax
import jax.numpy as jnp
from jax import lax
from jax.experimental import pallas as pl
from jax.experimental.pallas import tpu as pltpu
from jax.experimental.pallas import tpu_sc as plsc

_PAD = 1
_B = 4
_S = 8192
_E = 1024
_LANES = 16

_NC = 2
_NS = 16
_NW = _NC * _NS
_CHUNK = (_B * _S) // _NW
_CPR = _S // _CHUNK
_VPC = _CHUNK // _LANES
_BLK = 64
_NBLK = _CHUNK // _BLK
_VPB = _BLK // _LANES


def _body(inp_hbm, w_hbm, out_hbm, rowbuf, posbuf, fixbuf, lsems, fsem):
    c = lax.axis_index("c")
    s = lax.axis_index("s")
    wid = s * _NC + c
    row = wid // _CPR
    ci = wid % _CPR

    pltpu.sync_copy(inp_hbm.at[row], rowbuf)

    nvpre = ci * _VPC
    pad_v = jnp.full((_LANES,), _PAD, jnp.int32)
    one_v = jnp.full((_LANES,), 1, jnp.int32)

    def pre_step(j, vacc):
        v = rowbuf[pl.ds(j * _LANES, _LANES)]
        return vacc + jnp.where(v != pad_v, one_v, 0)

    vacc = lax.fori_loop(0, nvpre, pre_step, jnp.zeros((_LANES,), jnp.int32))
    carry_s = jnp.sum(vacc)
    base = wid * _CHUNK

    def lin_src(st):
        return w_hbm.at[pl.ds(st * _E, _BLK * _E)]

    def out_at(g):
        return out_hbm.at[pl.ds((base + g * _BLK) * _E, _BLK * _E)]

    starts = []
    nfix = jnp.int32(0)
    for g in range(_NBLK):
        st = carry_s + 1
        starts.append(st)
        pltpu.async_copy(lin_src(st), out_at(g), lsems.at[g])

        carry_v = jnp.broadcast_to(carry_s, (_LANES,))
        for j in range(_VPB):
            jj = (g * _VPB + j) * _LANES
            v = rowbuf[pl.ds(nvpre * _LANES + jj, _LANES)]
            m = jnp.where(v != pad_v, one_v, 0)
            cs = plsc.cumsum(m)
            posbuf[pl.ds(jj, _LANES)] = (cs + carry_v) * m + pad_v
            ms = jnp.sum(m)
            carry_s = carry_s + ms
            carry_v = carry_v + jnp.broadcast_to(ms, (_LANES,))

        bad = (carry_s - st + 1) != _BLK
        fixbuf[pl.ds(nfix * _LANES, _LANES)] = jnp.full((_LANES,), g,
                                                        jnp.int32)
        nfix = nfix + jnp.where(bad, 1, 0).astype(jnp.int32)

    for g in range(_NBLK):
        pltpu.make_async_copy(lin_src(starts[g]), out_at(g),
                              lsems.at[g]).wait()

    lane = plsc.cumsum(one_v) - one_v

    def fix_step(i, acc):
        gv = fixbuf[pl.ds(i * _LANES, _LANES)]
        g = jnp.max(gv)

        def row_step(t, acc2):
            tv = (t // _LANES) * _LANES
            tl = t % _LANES
            pv = posbuf[pl.ds(g * _BLK + tv, _LANES)]
            p = jnp.sum(jnp.where(lane == jnp.broadcast_to(tl, (_LANES,)),
                                  pv, 0))
            src = w_hbm.at[pl.ds(p * _E, _E)]
            dst = out_hbm.at[pl.ds((base + g * _BLK + t) * _E, _E)]
            pltpu.async_copy(src, dst, fsem)
            pltpu.make_async_copy(src, dst, fsem).wait()
            return acc2

        return lax.fori_loop(0, _BLK, row_step, acc)

    lax.fori_loop(0, nfix, fix_step, jnp.int32(0))


@jax.jit
def _sc_embed(inp, weight):
    mesh = plsc.VectorSubcoreMesh(core_axis_name="c", subcore_axis_name="s")
    out = pl.kernel(
        _body,
        out_type=jax.ShapeDtypeStruct((_B * _S * _E,), jnp.float32),
        mesh=mesh,
        compiler_params=pltpu.CompilerParams(needs_layout_passes=False),
        scratch_types=[
            pltpu.VMEM((_S,), jnp.int32),
            pltpu.VMEM((_CHUNK,), jnp.int32),
            pltpu.VMEM((_NBLK * _LANES,), jnp.int32),
            pltpu.SemaphoreType.DMA((_NBLK,)),
            pltpu.SemaphoreType.DMA,
        ],
    )(inp, weight.reshape(-1))
    return out


def kernel(input, weight):
    return _sc_embed(input, weight).reshape(_B, _S, _E)


# speedup vs baseline: 35.6207x; 35.6207x over previous
import jax
import jax.numpy as jnp
from jax import lax
from jax.experimental import pallas as pl
from jax.experimental.pallas import tpu as pltpu
from jax.experimental.pallas import tpu_sc as plsc

_PAD = 1
_B = 4
_S = 8192
_E = 1024
_LANES = 16

_NC = 2
_NS = 16
_NW = _NC * _NS
_CHUNK = (_B * _S) // _NW
_CPR = _S // _CHUNK
_VPC = _CHUNK // _LANES
_GROWS = 16
_NG = _CHUNK // _GROWS
_NBUF = 7


def _body(inp_hbm, w_hbm, out_hbm, rowbuf, posbuf, gbuf, gsems, wsems):
    c = lax.axis_index("c")
    s = lax.axis_index("s")
    wid = s * _NC + c
    row = wid // _CPR
    ci = wid % _CPR

    pltpu.sync_copy(inp_hbm.at[row], rowbuf)

    nvpre = ci * _VPC
    pad_v = jnp.full((_LANES,), _PAD, jnp.int32)
    one_v = jnp.full((_LANES,), 1, jnp.int32)

    def pre_step(j, vacc):
        v = rowbuf[pl.ds(j * _LANES, _LANES)]
        return vacc + jnp.where(v != pad_v, one_v, 0)

    vacc = lax.fori_loop(0, nvpre, pre_step, jnp.zeros((_LANES,), jnp.int32))
    carry0 = jnp.broadcast_to(jnp.sum(vacc), (_LANES,))

    def pos_step(j, carry):
        v = rowbuf[pl.ds((nvpre + j) * _LANES, _LANES)]
        m = jnp.where(v != pad_v, one_v, 0)
        cs = plsc.cumsum(m)
        posbuf[pl.ds(j * _LANES, _LANES)] = (cs + carry) * m + pad_v
        return carry + jnp.broadcast_to(jnp.sum(m), (_LANES,))

    lax.fori_loop(0, _VPC, pos_step, carry0)

    base = wid * _CHUNK

    def idx(g):
        return posbuf.at[pl.ds(g * _GROWS, _GROWS)]

    def buf(b):
        return gbuf.at[pl.ds(b * _GROWS, _GROWS)]

    def out_at(g):
        return out_hbm.at[pl.ds(base + g * _GROWS, _GROWS)]

    def start_gather(g, b):
        pltpu.async_copy(w_hbm.at[idx(g)], buf(b), gsems.at[b])

    def wait_gather(g, b):
        pltpu.make_async_copy(w_hbm.at[idx(g)], buf(b), gsems.at[b]).wait()

    def start_wb(g, b):
        pltpu.async_copy(buf(b), out_at(g), wsems.at[b])

    def wait_wb(g, b):
        pltpu.make_async_copy(buf(b), out_at(g), wsems.at[b]).wait()

    for g in range(_NBUF - 1):
        start_gather(g, g)

    for g in range(_NG):
        b = g % _NBUF
        wait_gather(g, b)
        start_wb(g, b)
        ng = g + _NBUF - 1
        nb = ng % _NBUF
        if g >= 1:
            wait_wb(g - 1, nb)
        if ng < _NG:
            start_gather(ng, nb)

    wait_wb(_NG - 1, (_NG - 1) % _NBUF)


@jax.jit
def _sc_embed(inp, weight):
    mesh = plsc.VectorSubcoreMesh(core_axis_name="c", subcore_axis_name="s")
    return pl.kernel(
        _body,
        out_type=jax.ShapeDtypeStruct((_B * _S, _E), jnp.float32),
        mesh=mesh,
        compiler_params=pltpu.CompilerParams(needs_layout_passes=False),
        scratch_types=[
            pltpu.VMEM((_S,), jnp.int32),
            pltpu.VMEM((_CHUNK,), jnp.int32),
            pltpu.VMEM((_NBUF * _GROWS, _E), jnp.float32),
            pltpu.SemaphoreType.DMA((_NBUF,)),
            pltpu.SemaphoreType.DMA((_NBUF,)),
        ],
    )(inp, weight)


def kernel(input, weight):
    return _sc_embed(input, weight).reshape(_B, _S, _E)
